# Initial kernel scaffold; baseline (speedup 1.0000x reference)
import jax
import jax.numpy as jnp
from jax.experimental import pallas as pl


def kernel(x, edge_index, edge_attr, batch, atom_emb, bond_emb, eps, W1, b1, g1, be1, W2, b2, g, be):
    raise NotImplementedError("write your pallas kernel here")


# speedup vs baseline: 4.9352x; 4.9352x over previous
import functools

import jax
import jax.numpy as jnp
from jax import lax
from jax.experimental import pallas as pl
from jax.experimental.pallas import tpu as pltpu
from jax.experimental.pallas import tpu_sc as plsc

N, E, D, L = 10000, 320000, 128, 2
NC, NS = 2, 16
NW = NC * NS
EPT = E // NW
C = 80
NP = 10240
RPT = NP // NS
ZR = 160


EB = 2000


def _encode_body(x_ref, atom_ref, bond_ref, h_ref, t_ref):
    h = jnp.zeros((EB, D), jnp.float32)
    ids64 = lax.broadcasted_iota(jnp.int32, (1, 64), 1)
    for i in range(9):
        col = x_ref[:, i][:, None]
        oh = (col == ids64).astype(jnp.float32)
        h = h + jnp.dot(oh, atom_ref[i],
                        preferred_element_type=jnp.float32,
                        precision=lax.Precision.HIGHEST)
    h_ref[...] = h

    @pl.when(pl.program_id(0) == 0)
    def _tables():
        cidx = lax.broadcasted_iota(jnp.int32, (512, 1), 0)
        ids8 = lax.broadcasted_iota(jnp.int32, (1, 8), 1)
        oh0 = ((cidx // 64) == ids8).astype(jnp.float32)
        oh1 = (((cidx // 8) % 8) == ids8).astype(jnp.float32)
        oh2 = ((cidx % 8) == ids8).astype(jnp.float32)
        for l in range(L):
            t = (jnp.dot(oh0, bond_ref[l, 0],
                         preferred_element_type=jnp.float32,
                         precision=lax.Precision.HIGHEST)
                 + jnp.dot(oh1, bond_ref[l, 1],
                           preferred_element_type=jnp.float32,
                           precision=lax.Precision.HIGHEST)
                 + jnp.dot(oh2, bond_ref[l, 2],
                           preferred_element_type=jnp.float32,
                           precision=lax.Precision.HIGHEST))
            t_ref[l] = t


_encode = pl.pallas_call(
    _encode_body,
    grid=(N // EB,),
    in_specs=[
        pl.BlockSpec((EB, 9), lambda i: (i, 0)),
        pl.BlockSpec((9, 64, D), lambda i: (0, 0, 0)),
        pl.BlockSpec((L, 3, 8, D), lambda i: (0, 0, 0, 0)),
    ],
    out_specs=(
        pl.BlockSpec((EB, D), lambda i: (i, 0)),
        pl.BlockSpec((L, 512, D), lambda i: (0, 0, 0)),
    ),
    out_shape=(
        jax.ShapeDtypeStruct((N, D), jnp.float32),
        jax.ShapeDtypeStruct((L, 512, D), jnp.float32),
    ),
)


def _edge_body(h_hbm, src_hbm, dst_hbm, a0_hbm, a1_hbm, a2_hbm, t_hbm,
               out_hbm,
               srcb, dstb, a0b, a1b, a2b, cb, hb, tb, zb, aggsh,
               sem1, sem2):
    cid = lax.axis_index("c")
    sid = lax.axis_index("s")

    zero16 = jnp.zeros((16,), jnp.float32)

    def _zrow(r, _):
        for q in range(D // 16):
            zb[r, pl.ds(q * 16, 16)] = zero16
        return 0
    lax.fori_loop(0, ZR, _zrow, 0)
    for t in range(RPT // ZR):
        r0 = sid * RPT + t * ZR
        pltpu.sync_copy(zb, aggsh.at[pl.ds(r0, ZR)])
    plsc.subcore_barrier()

    base = (cid * NS + sid) * EPT

    def _chunk(i, _):
        off = base + i * C
        pltpu.sync_copy(src_hbm.at[pl.ds(off, C)], srcb)
        pltpu.sync_copy(dst_hbm.at[pl.ds(off, C)], dstb)
        pltpu.sync_copy(a0_hbm.at[pl.ds(off, C)], a0b)
        pltpu.sync_copy(a1_hbm.at[pl.ds(off, C)], a1b)
        pltpu.sync_copy(a2_hbm.at[pl.ds(off, C)], a2b)
        for j in range(C // 16):
            sl = pl.ds(j * 16, 16)
            cb[sl] = a0b[sl] * 64 + a1b[sl] * 8 + a2b[sl]
        g1 = pltpu.async_copy(h_hbm.at[srcb], hb, sem1)
        g2 = pltpu.async_copy(t_hbm.at[cb], tb, sem2)
        g1.wait()
        g2.wait()

        def _row(r, _):
            for q in range(D // 16):
                sl = pl.ds(q * 16, 16)
                hb[r, sl] = jnp.maximum(hb[r, sl] + tb[r, sl], 0.0)
            return 0
        lax.fori_loop(0, C, _row, 0)
        pltpu.sync_copy(hb, aggsh.at[dstb], add=True)
        return 0

    lax.fori_loop(0, EPT // C, _chunk, 0)
    plsc.subcore_barrier()

    for t in range(RPT // ZR):
        r0 = sid * RPT + t * ZR
        pltpu.sync_copy(aggsh.at[pl.ds(r0, ZR)], out_hbm.at[cid, pl.ds(r0, ZR)])


_EDGE_CACHE = {}


def _build_edge_phase():
    return functools.partial(
        pl.kernel,
        out_type=jax.ShapeDtypeStruct((NC, NP, D), jnp.float32),
        mesh=plsc.VectorSubcoreMesh(core_axis_name="c", subcore_axis_name="s",
                                    num_cores=NC, num_subcores=NS),
        scratch_types=[
        pltpu.VMEM((C,), jnp.int32),
        pltpu.VMEM((C,), jnp.int32),
        pltpu.VMEM((C,), jnp.int32),
        pltpu.VMEM((C,), jnp.int32),
        pltpu.VMEM((C,), jnp.int32),
        pltpu.VMEM((C,), jnp.int32),
        pltpu.VMEM((C, D), jnp.float32),
        pltpu.VMEM((C, D), jnp.float32),
        pltpu.VMEM((ZR, D), jnp.float32),
        pltpu.VMEM_SHARED((NP, D), jnp.float32),
            pltpu.SemaphoreType.DMA,
            pltpu.SemaphoreType.DMA,
        ],
    )(_edge_body)


def _edge_phase(*args):
    if "f" not in _EDGE_CACHE:
        _EDGE_CACHE["f"] = _build_edge_phase()
    return _EDGE_CACHE["f"](*args)


def _mlp_body(h_ref, agg_ref, eps_ref, w1_ref, b1_ref, g1_ref, be1_ref,
              w2_ref, b2_ref, g_ref, be_ref, out_ref, *, last):
    z = (1.0 + eps_ref[0]) * h_ref[...] + agg_ref[0, :N] + agg_ref[1, :N]
    y = jnp.dot(z, w1_ref[...], preferred_element_type=jnp.float32)
    y = y + b1_ref[...]
    m = jnp.mean(y, axis=0, keepdims=True)
    v = jnp.mean((y - m) ** 2, axis=0, keepdims=True)
    y = (y - m) * lax.rsqrt(v + 1e-5) * g1_ref[...] + be1_ref[...]
    y = jnp.maximum(y, 0.0)
    o = jnp.dot(y, w2_ref[...], preferred_element_type=jnp.float32)
    o = o + b2_ref[...]
    m2 = jnp.mean(o, axis=0, keepdims=True)
    v2 = jnp.mean((o - m2) ** 2, axis=0, keepdims=True)
    o = (o - m2) * lax.rsqrt(v2 + 1e-5) * g_ref[...] + be_ref[...]
    if not last:
        o = jnp.maximum(o, 0.0)
    out_ref[...] = o


def _mlp(last):
    return pl.pallas_call(
        functools.partial(_mlp_body, last=last),
        in_specs=[
            pl.BlockSpec(memory_space=pltpu.VMEM),
            pl.BlockSpec(memory_space=pltpu.VMEM),
            pl.BlockSpec(memory_space=pltpu.SMEM),
            pl.BlockSpec(memory_space=pltpu.VMEM),
            pl.BlockSpec(memory_space=pltpu.VMEM),
            pl.BlockSpec(memory_space=pltpu.VMEM),
            pl.BlockSpec(memory_space=pltpu.VMEM),
            pl.BlockSpec(memory_space=pltpu.VMEM),
            pl.BlockSpec(memory_space=pltpu.VMEM),
            pl.BlockSpec(memory_space=pltpu.VMEM),
            pl.BlockSpec(memory_space=pltpu.VMEM),
        ],
        out_shape=jax.ShapeDtypeStruct((N, D), jnp.float32),
    )


def kernel(x, edge_index, edge_attr, batch, atom_emb, bond_emb, eps,
           W1, b1, g1, be1, W2, b2, g, be):
    h, T = _encode(x.astype(jnp.int32), atom_emb, bond_emb)
    src = edge_index[0].astype(jnp.int32)
    dst = edge_index[1].astype(jnp.int32)
    ea = edge_attr.astype(jnp.int32)
    a0, a1, a2 = ea[:, 0], ea[:, 1], ea[:, 2]
    for l in range(L):
        agg = _edge_phase(h, src, dst, a0, a1, a2, T[l])
        h = _mlp(l == L - 1)(
            h, agg, eps[l].reshape(1),
            W1[l], b1[l].reshape(1, 2 * D), g1[l].reshape(1, 2 * D),
            be1[l].reshape(1, 2 * D),
            W2[l], b2[l].reshape(1, D), g[l].reshape(1, D),
            be[l].reshape(1, D))
    return (h, batch)


# speedup vs baseline: 10.5663x; 2.1410x over previous
import functools

import jax
import jax.numpy as jnp
from jax import lax
from jax.experimental import pallas as pl
from jax.experimental.pallas import tpu as pltpu
from jax.experimental.pallas import tpu_sc as plsc

N, E, D, L = 10000, 320000, 128, 2
DH = D // 2
NC, NS = 2, 16
EPT = E // NS
C = 80
NP = 10240
RPT = NP // NS
ZR = 160


EB = 2000


def _encode_body(x_ref, atom_ref, bond_ref, h_ref, t_ref):
    h = jnp.zeros((EB, D), jnp.float32)
    ids64 = lax.broadcasted_iota(jnp.int32, (1, 64), 1)
    for i in range(9):
        col = x_ref[:, i][:, None]
        oh = (col == ids64).astype(jnp.float32)
        h = h + jnp.dot(oh, atom_ref[i],
                        preferred_element_type=jnp.float32,
                        precision=lax.Precision.HIGHEST)
    h_ref[0] = h[:, :DH]
    h_ref[1] = h[:, DH:]

    @pl.when(pl.program_id(0) == 0)
    def _tables():
        cidx = lax.broadcasted_iota(jnp.int32, (512, 1), 0)
        ids8 = lax.broadcasted_iota(jnp.int32, (1, 8), 1)
        oh0 = ((cidx // 64) == ids8).astype(jnp.float32)
        oh1 = (((cidx // 8) % 8) == ids8).astype(jnp.float32)
        oh2 = ((cidx % 8) == ids8).astype(jnp.float32)
        for l in range(L):
            t = (jnp.dot(oh0, bond_ref[l, 0],
                         preferred_element_type=jnp.float32,
                         precision=lax.Precision.HIGHEST)
                 + jnp.dot(oh1, bond_ref[l, 1],
                           preferred_element_type=jnp.float32,
                           precision=lax.Precision.HIGHEST)
                 + jnp.dot(oh2, bond_ref[l, 2],
                           preferred_element_type=jnp.float32,
                           precision=lax.Precision.HIGHEST))
            t_ref[l, 0] = t[:, :DH]
            t_ref[l, 1] = t[:, DH:]


_encode = pl.pallas_call(
    _encode_body,
    grid=(N // EB,),
    in_specs=[
        pl.BlockSpec((EB, 9), lambda i: (i, 0)),
        pl.BlockSpec((9, 64, D), lambda i: (0, 0, 0)),
        pl.BlockSpec((L, 3, 8, D), lambda i: (0, 0, 0, 0)),
    ],
    out_specs=(
        pl.BlockSpec((2, EB, DH), lambda i: (0, i, 0)),
        pl.BlockSpec((L, 2, 512, DH), lambda i: (0, 0, 0, 0)),
    ),
    out_shape=(
        jax.ShapeDtypeStruct((2, N, DH), jnp.float32),
        jax.ShapeDtypeStruct((L, 2, 512, DH), jnp.float32),
    ),
)


NCH = EPT // C


def _edge_body(h_hbm, src_hbm, dst_hbm, a0_hbm, a1_hbm, a2_hbm, t_hbm,
               out_hbm,
               a0b0, a0b1, a1b0, a1b1, a2b0, a2b1,
               srcb0, srcb1, dstb0, dstb1, dstS0, dstS1,
               cb0, cb1, hb0, hb1, tb0, tb1, mb0, mb1, zb, aggsh,
               semi0, semi1, semh0, semh1, semt0, semt1, sems0, sems1):
    cid = lax.axis_index("c")
    sid = lax.axis_index("s")
    base = sid * EPT

    slots = (
        ((a0b0, a1b0, a2b0), srcb0, dstb0, dstS0, cb0, hb0, tb0, mb0,
         semi0, semh0, semt0, sems0),
        ((a0b1, a1b1, a2b1), srcb1, dstb1, dstS1, cb1, hb1, tb1, mb1,
         semi1, semh1, semt1, sems1),
    )

    def idx_issue(k, s):
        (a0b_, a1b_, a2b_), srcb_, dstb_ = (slots[s][0], slots[s][1],
                                            slots[s][2])
        semi_ = slots[s][8]
        off = base + k * C
        pltpu.async_copy(src_hbm.at[pl.ds(off, C)], srcb_, semi_)
        pltpu.async_copy(dst_hbm.at[pl.ds(off, C)], dstb_, semi_)
        pltpu.async_copy(a0_hbm.at[pl.ds(off, C)], a0b_, semi_)
        pltpu.async_copy(a1_hbm.at[pl.ds(off, C)], a1b_, semi_)
        pltpu.async_copy(a2_hbm.at[pl.ds(off, C)], a2b_, semi_)

    def idx_wait(k, s):
        (a0b_, a1b_, a2b_), srcb_, dstb_ = (slots[s][0], slots[s][1],
                                            slots[s][2])
        semi_ = slots[s][8]
        off = base + k * C
        pltpu.make_async_copy(src_hbm.at[pl.ds(off, C)], srcb_, semi_).wait()
        pltpu.make_async_copy(dst_hbm.at[pl.ds(off, C)], dstb_, semi_).wait()
        pltpu.make_async_copy(a0_hbm.at[pl.ds(off, C)], a0b_, semi_).wait()
        pltpu.make_async_copy(a1_hbm.at[pl.ds(off, C)], a1b_, semi_).wait()
        pltpu.make_async_copy(a2_hbm.at[pl.ds(off, C)], a2b_, semi_).wait()

    def cidx(s):
        (a0b_, a1b_, a2b_), srcb_, cb_ = (slots[s][0], slots[s][1],
                                          slots[s][4])
        coff = cid * 512
        soff = cid * N
        for g in range(C // 16):
            sl = pl.ds(g * 16, 16)
            cb_[sl] = a0b_[sl] * 64 + a1b_[sl] * 8 + a2b_[sl] + coff
            srcb_[sl] = srcb_[sl] + soff

    def gather_issue(s):
        srcb_, cb_, hb_, tb_ = (slots[s][1], slots[s][4], slots[s][5],
                                slots[s][6])
        semh_, semt_ = slots[s][9], slots[s][10]
        pltpu.async_copy(h_hbm.at[srcb_], hb_, semh_)
        pltpu.async_copy(t_hbm.at[cb_], tb_, semt_)

    def gather_wait(s):
        srcb_, cb_, hb_, tb_ = (slots[s][1], slots[s][4], slots[s][5],
                                slots[s][6])
        semh_, semt_ = slots[s][9], slots[s][10]
        pltpu.make_async_copy(h_hbm.at[srcb_], hb_, semh_).wait()
        pltpu.make_async_copy(t_hbm.at[cb_], tb_, semt_).wait()

    def dst_snap(s):
        dstb_, dstS_ = slots[s][2], slots[s][3]
        for g in range(C // 16):
            sl = pl.ds(g * 16, 16)
            dstS_[sl] = dstb_[sl]

    def compute(s):
        hb_, tb_, mb_ = slots[s][5], slots[s][6], slots[s][7]

        def _row(r, _):
            for q in range(DH // 16):
                sl = pl.ds(q * 16, 16)
                mb_[r, sl] = jnp.maximum(hb_[r, sl] + tb_[r, sl], 0.0)
            return 0
        lax.fori_loop(0, C, _row, 0)

    def scat_issue(s):
        dstS_, mb_ = slots[s][3], slots[s][7]
        pltpu.sync_copy(mb_, aggsh.at[dstS_], add=True)

    zero16 = jnp.zeros((16,), jnp.float32)

    def _zrow(r, _):
        for q in range(DH // 16):
            zb[r, pl.ds(q * 16, 16)] = zero16
        return 0
    lax.fori_loop(0, ZR, _zrow, 0)
    for t in range(RPT // ZR):
        r0 = sid * RPT + t * ZR
        pltpu.sync_copy(zb, aggsh.at[pl.ds(r0, ZR)])
    plsc.subcore_barrier()

    idx_issue(0, 0)
    idx_wait(0, 0)
    cidx(0)
    gather_issue(0)
    idx_issue(1, 1)

    def _pair(j, _):
        for s in (0, 1):
            k = 2 * j + s

            @pl.when(k + 1 < NCH)
            def _():
                idx_wait(k + 1, 1 - s)
                cidx(1 - s)
                gather_issue(1 - s)
            gather_wait(s)
            dst_snap(s)

            @pl.when(k + 2 < NCH)
            def _():
                idx_issue(k + 2, s)
            compute(s)
            scat_issue(s)
        return 0

    lax.fori_loop(0, NCH // 2, _pair, 0)
    plsc.subcore_barrier()

    for t in range(RPT // ZR):
        r0 = sid * RPT + t * ZR
        pltpu.sync_copy(aggsh.at[pl.ds(r0, ZR)], out_hbm.at[cid, pl.ds(r0, ZR)])


_EDGE_CACHE = {}


def _build_edge_phase():
    return functools.partial(
        pl.kernel,
        out_type=jax.ShapeDtypeStruct((NC, NP, DH), jnp.float32),
        mesh=plsc.VectorSubcoreMesh(core_axis_name="c", subcore_axis_name="s",
                                    num_cores=NC, num_subcores=NS),
        compiler_params=pltpu.CompilerParams(use_tc_tiling_on_sc=False),
        scratch_types=(
            [pltpu.VMEM((C,), jnp.int32)] * 6
            + [pltpu.VMEM((C,), jnp.int32)] * 6
            + [pltpu.VMEM((C,), jnp.int32)] * 2
            + [pltpu.VMEM((C, DH), jnp.float32)] * 6
            + [pltpu.VMEM((ZR, DH), jnp.float32)]
            + [pltpu.VMEM_SHARED((NP, DH), jnp.float32)]
            + [pltpu.SemaphoreType.DMA] * 8
        ),
    )(_edge_body)


def _edge_phase(*args):
    if "f" not in _EDGE_CACHE:
        _EDGE_CACHE["f"] = _build_edge_phase()
    return _EDGE_CACHE["f"](*args)


def _mlp_body(h_ref, agg_ref, eps_ref, w1_ref, b1_ref, g1_ref, be1_ref,
              w2_ref, b2_ref, g_ref, be_ref, out_ref, *, last):
    hfull = jnp.concatenate([h_ref[0], h_ref[1]], axis=1)
    afull = jnp.concatenate([agg_ref[0, :N], agg_ref[1, :N]], axis=1)
    z = (1.0 + eps_ref[0]) * hfull + afull
    y = jnp.dot(z, w1_ref[...], preferred_element_type=jnp.float32)
    y = y + b1_ref[...]
    m = jnp.mean(y, axis=0, keepdims=True)
    v = jnp.mean((y - m) ** 2, axis=0, keepdims=True)
    y = (y - m) * lax.rsqrt(v + 1e-5) * g1_ref[...] + be1_ref[...]
    y = jnp.maximum(y, 0.0)
    o = jnp.dot(y, w2_ref[...], preferred_element_type=jnp.float32)
    o = o + b2_ref[...]
    m2 = jnp.mean(o, axis=0, keepdims=True)
    v2 = jnp.mean((o - m2) ** 2, axis=0, keepdims=True)
    o = (o - m2) * lax.rsqrt(v2 + 1e-5) * g_ref[...] + be_ref[...]
    if last:
        out_ref[...] = o
    else:
        o = jnp.maximum(o, 0.0)
        out_ref[0] = o[:, :DH]
        out_ref[1] = o[:, DH:]


def _mlp(last):
    return pl.pallas_call(
        functools.partial(_mlp_body, last=last),
        in_specs=[
            pl.BlockSpec(memory_space=pltpu.VMEM),
            pl.BlockSpec(memory_space=pltpu.VMEM),
            pl.BlockSpec(memory_space=pltpu.SMEM),
            pl.BlockSpec(memory_space=pltpu.VMEM),
            pl.BlockSpec(memory_space=pltpu.VMEM),
            pl.BlockSpec(memory_space=pltpu.VMEM),
            pl.BlockSpec(memory_space=pltpu.VMEM),
            pl.BlockSpec(memory_space=pltpu.VMEM),
            pl.BlockSpec(memory_space=pltpu.VMEM),
            pl.BlockSpec(memory_space=pltpu.VMEM),
            pl.BlockSpec(memory_space=pltpu.VMEM),
        ],
        out_shape=(jax.ShapeDtypeStruct((N, D), jnp.float32) if last
                   else jax.ShapeDtypeStruct((2, N, DH), jnp.float32)),
    )


def kernel(x, edge_index, edge_attr, batch, atom_emb, bond_emb, eps,
           W1, b1, g1, be1, W2, b2, g, be):
    h, T = _encode(x.astype(jnp.int32), atom_emb, bond_emb)
    src = edge_index[0].astype(jnp.int32)
    dst = edge_index[1].astype(jnp.int32)
    ea = edge_attr.astype(jnp.int32)
    a0, a1, a2 = ea[:, 0], ea[:, 1], ea[:, 2]
    for l in range(L):
        agg = _edge_phase(h.reshape(2 * N, DH), src, dst, a0, a1, a2,
                          T[l].reshape(2 * 512, DH))
        h = _mlp(l == L - 1)(
            h, agg, eps[l].reshape(1),
            W1[l], b1[l].reshape(1, 2 * D), g1[l].reshape(1, 2 * D),
            be1[l].reshape(1, 2 * D),
            W2[l], b2[l].reshape(1, D), g[l].reshape(1, D),
            be[l].reshape(1, D))
    return (h, batch)


# speedup vs baseline: 10.6112x; 1.0042x over previous
import functools

import jax
import jax.numpy as jnp
from jax import lax
from jax.experimental import pallas as pl
from jax.experimental.pallas import tpu as pltpu
from jax.experimental.pallas import tpu_sc as plsc

N, E, D, L = 10000, 320000, 128, 2
DH = D // 2
NC, NS = 2, 16
EPT = E // NS
C = 80
NP = 10240
RPT = NP // NS
ZR = 160


EB = 2000


def _encode_body(x_ref, atom_ref, bond_ref, h_ref, t_ref):
    h = jnp.zeros((EB, D), jnp.float32)
    ids64 = lax.broadcasted_iota(jnp.int32, (1, 64), 1)
    for i in range(9):
        col = x_ref[:, i][:, None]
        oh = (col == ids64).astype(jnp.float32)
        h = h + jnp.dot(oh, atom_ref[i],
                        preferred_element_type=jnp.float32,
                        precision=lax.Precision.HIGHEST)
    h_ref[0] = h[:, :DH]
    h_ref[1] = h[:, DH:]

    @pl.when(pl.program_id(0) == 0)
    def _tables():
        cidx = lax.broadcasted_iota(jnp.int32, (512, 1), 0)
        ids8 = lax.broadcasted_iota(jnp.int32, (1, 8), 1)
        oh0 = ((cidx // 64) == ids8).astype(jnp.float32)
        oh1 = (((cidx // 8) % 8) == ids8).astype(jnp.float32)
        oh2 = ((cidx % 8) == ids8).astype(jnp.float32)
        for l in range(L):
            t = (jnp.dot(oh0, bond_ref[l, 0],
                         preferred_element_type=jnp.float32,
                         precision=lax.Precision.HIGHEST)
                 + jnp.dot(oh1, bond_ref[l, 1],
                           preferred_element_type=jnp.float32,
                           precision=lax.Precision.HIGHEST)
                 + jnp.dot(oh2, bond_ref[l, 2],
                           preferred_element_type=jnp.float32,
                           precision=lax.Precision.HIGHEST))
            t_ref[l, 0] = t[:, :DH]
            t_ref[l, 1] = t[:, DH:]


_encode = pl.pallas_call(
    _encode_body,
    grid=(N // EB,),
    in_specs=[
        pl.BlockSpec((EB, 9), lambda i: (i, 0)),
        pl.BlockSpec((9, 64, D), lambda i: (0, 0, 0)),
        pl.BlockSpec((L, 3, 8, D), lambda i: (0, 0, 0, 0)),
    ],
    out_specs=(
        pl.BlockSpec((2, EB, DH), lambda i: (0, i, 0)),
        pl.BlockSpec((L, 2, 512, DH), lambda i: (0, 0, 0, 0)),
    ),
    out_shape=(
        jax.ShapeDtypeStruct((2, N, DH), jnp.float32),
        jax.ShapeDtypeStruct((L, 2, 512, DH), jnp.float32),
    ),
)


NCH = EPT // C


def _edge_body(h_hbm, src_hbm, dst_hbm, a0_hbm, a1_hbm, a2_hbm, t_hbm,
               out_hbm,
               a0b0, a0b1, a1b0, a1b1, a2b0, a2b1,
               srcb0, srcb1, dstb0, dstb1, dstS0, dstS1,
               cb0, cb1, hb0, hb1, tb0, tb1, mb0, mb1, zb, aggsh,
               semi0, semi1, semh0, semh1, semt0, semt1, sems0, sems1):
    cid = lax.axis_index("c")
    sid = lax.axis_index("s")
    base = sid * EPT

    slots = (
        ((a0b0, a1b0, a2b0), srcb0, dstb0, dstS0, cb0, hb0, tb0, mb0,
         semi0, semh0, semt0, sems0),
        ((a0b1, a1b1, a2b1), srcb1, dstb1, dstS1, cb1, hb1, tb1, mb1,
         semi1, semh1, semt1, sems1),
    )

    def idx_issue(k, s):
        (a0b_, a1b_, a2b_), srcb_, dstb_ = (slots[s][0], slots[s][1],
                                            slots[s][2])
        semi_ = slots[s][8]
        off = base + k * C
        pltpu.async_copy(src_hbm.at[pl.ds(off, C)], srcb_, semi_)
        pltpu.async_copy(dst_hbm.at[pl.ds(off, C)], dstb_, semi_)
        pltpu.async_copy(a0_hbm.at[pl.ds(off, C)], a0b_, semi_)
        pltpu.async_copy(a1_hbm.at[pl.ds(off, C)], a1b_, semi_)
        pltpu.async_copy(a2_hbm.at[pl.ds(off, C)], a2b_, semi_)

    def idx_wait(k, s):
        (a0b_, a1b_, a2b_), srcb_, dstb_ = (slots[s][0], slots[s][1],
                                            slots[s][2])
        semi_ = slots[s][8]
        off = base + k * C
        pltpu.make_async_copy(src_hbm.at[pl.ds(off, C)], srcb_, semi_).wait()
        pltpu.make_async_copy(dst_hbm.at[pl.ds(off, C)], dstb_, semi_).wait()
        pltpu.make_async_copy(a0_hbm.at[pl.ds(off, C)], a0b_, semi_).wait()
        pltpu.make_async_copy(a1_hbm.at[pl.ds(off, C)], a1b_, semi_).wait()
        pltpu.make_async_copy(a2_hbm.at[pl.ds(off, C)], a2b_, semi_).wait()

    def cidx(s):
        (a0b_, a1b_, a2b_), srcb_, cb_ = (slots[s][0], slots[s][1],
                                          slots[s][4])
        coff = cid * 512
        soff = cid * N
        for g in range(C // 16):
            sl = pl.ds(g * 16, 16)
            cb_[sl] = a0b_[sl] * 64 + a1b_[sl] * 8 + a2b_[sl] + coff
            srcb_[sl] = srcb_[sl] + soff

    def gather_issue(s):
        srcb_, cb_, hb_, tb_ = (slots[s][1], slots[s][4], slots[s][5],
                                slots[s][6])
        semh_, semt_ = slots[s][9], slots[s][10]
        pltpu.async_copy(h_hbm.at[srcb_], hb_, semh_)
        pltpu.async_copy(t_hbm.at[cb_], tb_, semt_)

    def gather_wait(s):
        srcb_, cb_, hb_, tb_ = (slots[s][1], slots[s][4], slots[s][5],
                                slots[s][6])
        semh_, semt_ = slots[s][9], slots[s][10]
        pltpu.make_async_copy(h_hbm.at[srcb_], hb_, semh_).wait()
        pltpu.make_async_copy(t_hbm.at[cb_], tb_, semt_).wait()

    def dst_snap(s):
        dstb_, dstS_ = slots[s][2], slots[s][3]
        for g in range(C // 16):
            sl = pl.ds(g * 16, 16)
            dstS_[sl] = dstb_[sl]

    def compute(s):
        hb_, tb_, mb_ = slots[s][5], slots[s][6], slots[s][7]

        def _row(r, _):
            for q in range(DH // 16):
                sl = pl.ds(q * 16, 16)
                mb_[r, sl] = jnp.maximum(hb_[r, sl] + tb_[r, sl], 0.0)
            return 0
        lax.fori_loop(0, C, _row, 0)

    def scat_issue(s):
        dstS_, mb_, sems_ = slots[s][3], slots[s][7], slots[s][11]
        pltpu.async_copy(mb_, aggsh.at[dstS_], sems_, add=True)

    def scat_wait(s):
        dstS_, mb_, sems_ = slots[s][3], slots[s][7], slots[s][11]
        pltpu.make_async_copy(mb_, aggsh.at[dstS_], sems_).wait()

    zero16 = jnp.zeros((16,), jnp.float32)

    def _zrow(r, _):
        for q in range(DH // 16):
            zb[r, pl.ds(q * 16, 16)] = zero16
        return 0
    lax.fori_loop(0, ZR, _zrow, 0)
    for t in range(RPT // ZR):
        r0 = sid * RPT + t * ZR
        pltpu.sync_copy(zb, aggsh.at[pl.ds(r0, ZR)])
    plsc.subcore_barrier()

    idx_issue(0, 0)
    idx_wait(0, 0)
    cidx(0)
    gather_issue(0)
    idx_issue(1, 1)

    def _pair(j, _):
        for s in (0, 1):
            k = 2 * j + s

            @pl.when(k + 1 < NCH)
            def _():
                idx_wait(k + 1, 1 - s)
                cidx(1 - s)
                gather_issue(1 - s)
            gather_wait(s)

            @pl.when(k >= 2)
            def _():
                scat_wait(s)
            dst_snap(s)

            @pl.when(k + 2 < NCH)
            def _():
                idx_issue(k + 2, s)
            compute(s)
            scat_issue(s)
        return 0

    lax.fori_loop(0, NCH // 2, _pair, 0)
    scat_wait(0)
    scat_wait(1)
    plsc.subcore_barrier()

    for t in range(RPT // ZR):
        r0 = sid * RPT + t * ZR
        pltpu.sync_copy(aggsh.at[pl.ds(r0, ZR)], out_hbm.at[cid, pl.ds(r0, ZR)])


_EDGE_CACHE = {}


def _build_edge_phase():
    return functools.partial(
        pl.kernel,
        out_type=jax.ShapeDtypeStruct((NC, NP, DH), jnp.float32),
        mesh=plsc.VectorSubcoreMesh(core_axis_name="c", subcore_axis_name="s",
                                    num_cores=NC, num_subcores=NS),
        compiler_params=pltpu.CompilerParams(use_tc_tiling_on_sc=False),
        scratch_types=(
            [pltpu.VMEM((C,), jnp.int32)] * 6
            + [pltpu.VMEM((C,), jnp.int32)] * 6
            + [pltpu.VMEM((C,), jnp.int32)] * 2
            + [pltpu.VMEM((C, DH), jnp.float32)] * 6
            + [pltpu.VMEM((ZR, DH), jnp.float32)]
            + [pltpu.VMEM_SHARED((NP, DH), jnp.float32)]
            + [pltpu.SemaphoreType.DMA] * 8
        ),
    )(_edge_body)


def _edge_phase(*args):
    if "f" not in _EDGE_CACHE:
        _EDGE_CACHE["f"] = _build_edge_phase()
    return _EDGE_CACHE["f"](*args)


def _mlp_body(h_ref, agg_ref, eps_ref, w1_ref, b1_ref, g1_ref, be1_ref,
              w2_ref, b2_ref, g_ref, be_ref, out_ref, *, last):
    hfull = jnp.concatenate([h_ref[0], h_ref[1]], axis=1)
    afull = jnp.concatenate([agg_ref[0, :N], agg_ref[1, :N]], axis=1)
    z = (1.0 + eps_ref[0]) * hfull + afull
    y = jnp.dot(z, w1_ref[...], preferred_element_type=jnp.float32)
    y = y + b1_ref[...]
    m = jnp.mean(y, axis=0, keepdims=True)
    v = jnp.mean((y - m) ** 2, axis=0, keepdims=True)
    y = (y - m) * lax.rsqrt(v + 1e-5) * g1_ref[...] + be1_ref[...]
    y = jnp.maximum(y, 0.0)
    o = jnp.dot(y, w2_ref[...], preferred_element_type=jnp.float32)
    o = o + b2_ref[...]
    m2 = jnp.mean(o, axis=0, keepdims=True)
    v2 = jnp.mean((o - m2) ** 2, axis=0, keepdims=True)
    o = (o - m2) * lax.rsqrt(v2 + 1e-5) * g_ref[...] + be_ref[...]
    if last:
        out_ref[...] = o
    else:
        o = jnp.maximum(o, 0.0)
        out_ref[0] = o[:, :DH]
        out_ref[1] = o[:, DH:]


def _mlp(last):
    return pl.pallas_call(
        functools.partial(_mlp_body, last=last),
        in_specs=[
            pl.BlockSpec(memory_space=pltpu.VMEM),
            pl.BlockSpec(memory_space=pltpu.VMEM),
            pl.BlockSpec(memory_space=pltpu.SMEM),
            pl.BlockSpec(memory_space=pltpu.VMEM),
            pl.BlockSpec(memory_space=pltpu.VMEM),
            pl.BlockSpec(memory_space=pltpu.VMEM),
            pl.BlockSpec(memory_space=pltpu.VMEM),
            pl.BlockSpec(memory_space=pltpu.VMEM),
            pl.BlockSpec(memory_space=pltpu.VMEM),
            pl.BlockSpec(memory_space=pltpu.VMEM),
            pl.BlockSpec(memory_space=pltpu.VMEM),
        ],
        out_shape=(jax.ShapeDtypeStruct((N, D), jnp.float32) if last
                   else jax.ShapeDtypeStruct((2, N, DH), jnp.float32)),
    )


def kernel(x, edge_index, edge_attr, batch, atom_emb, bond_emb, eps,
           W1, b1, g1, be1, W2, b2, g, be):
    h, T = _encode(x.astype(jnp.int32), atom_emb, bond_emb)
    src = edge_index[0].astype(jnp.int32)
    dst = edge_index[1].astype(jnp.int32)
    ea = edge_attr.astype(jnp.int32)
    a0, a1, a2 = ea[:, 0], ea[:, 1], ea[:, 2]
    for l in range(L):
        agg = _edge_phase(h.reshape(2 * N, DH), src, dst, a0, a1, a2,
                          T[l].reshape(2 * 512, DH))
        h = _mlp(l == L - 1)(
            h, agg, eps[l].reshape(1),
            W1[l], b1[l].reshape(1, 2 * D), g1[l].reshape(1, 2 * D),
            be1[l].reshape(1, 2 * D),
            W2[l], b2[l].reshape(1, D), g[l].reshape(1, D),
            be[l].reshape(1, D))
    return (h, batch)


# speedup vs baseline: 10.8280x; 1.0204x over previous
import functools

import jax
import jax.numpy as jnp
from jax import lax
from jax.experimental import pallas as pl
from jax.experimental.pallas import tpu as pltpu
from jax.experimental.pallas import tpu_sc as plsc

N, E, D, L = 10000, 320000, 128, 2
DH = D // 2
NC, NS = 2, 16
EPT = E // NS
C = 80
NP = 10240
RPT = NP // NS
ZR = 160


EB = 2000


def _encode_body(x_ref, atom_ref, bond_ref, ea_ref, h_ref, t_ref, c_ref):
    c_ref[0, 0] = ea_ref[0] * 64 + ea_ref[1] * 8 + ea_ref[2]
    h = jnp.zeros((EB, D), jnp.float32)
    ids64 = lax.broadcasted_iota(jnp.int32, (1, 64), 1)
    for i in range(9):
        col = x_ref[:, i][:, None]
        oh = (col == ids64).astype(jnp.float32)
        h = h + jnp.dot(oh, atom_ref[i],
                        preferred_element_type=jnp.float32,
                        precision=lax.Precision.HIGHEST)
    h_ref[0] = h[:, :DH]
    h_ref[1] = h[:, DH:]

    @pl.when(pl.program_id(0) == 0)
    def _tables():
        cidx = lax.broadcasted_iota(jnp.int32, (512, 1), 0)
        ids8 = lax.broadcasted_iota(jnp.int32, (1, 8), 1)
        oh0 = ((cidx // 64) == ids8).astype(jnp.float32)
        oh1 = (((cidx // 8) % 8) == ids8).astype(jnp.float32)
        oh2 = ((cidx % 8) == ids8).astype(jnp.float32)
        for l in range(L):
            t = (jnp.dot(oh0, bond_ref[l, 0],
                         preferred_element_type=jnp.float32,
                         precision=lax.Precision.HIGHEST)
                 + jnp.dot(oh1, bond_ref[l, 1],
                           preferred_element_type=jnp.float32,
                           precision=lax.Precision.HIGHEST)
                 + jnp.dot(oh2, bond_ref[l, 2],
                           preferred_element_type=jnp.float32,
                           precision=lax.Precision.HIGHEST))
            t_ref[l, 0] = t[:, :DH]
            t_ref[l, 1] = t[:, DH:]


_encode = pl.pallas_call(
    _encode_body,
    grid=(N // EB,),
    in_specs=[
        pl.BlockSpec((EB, 9), lambda i: (i, 0)),
        pl.BlockSpec((9, 64, D), lambda i: (0, 0, 0)),
        pl.BlockSpec((L, 3, 8, D), lambda i: (0, 0, 0, 0)),
        pl.BlockSpec((3, E // (N // EB)), lambda i: (0, i)),
    ],
    out_specs=(
        pl.BlockSpec((2, EB, DH), lambda i: (0, i, 0)),
        pl.BlockSpec((L, 2, 512, DH), lambda i: (0, 0, 0, 0)),
        pl.BlockSpec((1, 1, E // (N // EB)), lambda i: (i, 0, 0)),
    ),
    out_shape=(
        jax.ShapeDtypeStruct((2, N, DH), jnp.float32),
        jax.ShapeDtypeStruct((L, 2, 512, DH), jnp.float32),
        jax.ShapeDtypeStruct((N // EB, 1, E // (N // EB)), jnp.int32),
    ),
)


NCH = EPT // C


def _edge_body(h_hbm, src_hbm, dst_hbm, c_hbm, t_hbm,
               out_hbm,
               srcb0, srcb1, dstb0, dstb1, dstS0, dstS1,
               cb0, cb1, hb0, hb1, tb0, tb1, mb0, mb1, zb, aggsh,
               semi0, semi1, semh0, semh1, semt0, semt1, sems0, sems1):
    cid = lax.axis_index("c")
    sid = lax.axis_index("s")
    base = sid * EPT

    slots = (
        (None, srcb0, dstb0, dstS0, cb0, hb0, tb0, mb0,
         semi0, semh0, semt0, sems0),
        (None, srcb1, dstb1, dstS1, cb1, hb1, tb1, mb1,
         semi1, semh1, semt1, sems1),
    )

    def idx_issue(k, s):
        srcb_, dstb_, cb_ = slots[s][1], slots[s][2], slots[s][4]
        semi_ = slots[s][8]
        off = base + k * C
        pltpu.async_copy(src_hbm.at[pl.ds(off, C)], srcb_, semi_)
        pltpu.async_copy(dst_hbm.at[pl.ds(off, C)], dstb_, semi_)
        pltpu.async_copy(c_hbm.at[pl.ds(off, C)], cb_, semi_)

    def idx_wait(k, s):
        srcb_, dstb_, cb_ = slots[s][1], slots[s][2], slots[s][4]
        semi_ = slots[s][8]
        off = base + k * C
        pltpu.make_async_copy(src_hbm.at[pl.ds(off, C)], srcb_, semi_).wait()
        pltpu.make_async_copy(dst_hbm.at[pl.ds(off, C)], dstb_, semi_).wait()
        pltpu.make_async_copy(c_hbm.at[pl.ds(off, C)], cb_, semi_).wait()

    def cidx(s):
        srcb_, cb_ = slots[s][1], slots[s][4]
        coff = cid * 512
        soff = cid * N
        for g in range(C // 16):
            sl = pl.ds(g * 16, 16)
            cb_[sl] = cb_[sl] + coff
            srcb_[sl] = srcb_[sl] + soff

    def gather_issue(s):
        srcb_, cb_, hb_, tb_ = (slots[s][1], slots[s][4], slots[s][5],
                                slots[s][6])
        semh_, semt_ = slots[s][9], slots[s][10]
        pltpu.async_copy(h_hbm.at[srcb_], hb_, semh_)
        pltpu.async_copy(t_hbm.at[cb_], tb_, semt_)

    def gather_wait(s):
        srcb_, cb_, hb_, tb_ = (slots[s][1], slots[s][4], slots[s][5],
                                slots[s][6])
        semh_, semt_ = slots[s][9], slots[s][10]
        pltpu.make_async_copy(h_hbm.at[srcb_], hb_, semh_).wait()
        pltpu.make_async_copy(t_hbm.at[cb_], tb_, semt_).wait()

    def dst_snap(s):
        dstb_, dstS_ = slots[s][2], slots[s][3]
        for g in range(C // 16):
            sl = pl.ds(g * 16, 16)
            dstS_[sl] = dstb_[sl]

    def compute(s):
        hb_, tb_, mb_ = slots[s][5], slots[s][6], slots[s][7]

        def _row(r, _):
            for q in range(DH // 16):
                sl = pl.ds(q * 16, 16)
                mb_[r, sl] = jnp.maximum(hb_[r, sl] + tb_[r, sl], 0.0)
            return 0
        lax.fori_loop(0, C, _row, 0)

    def scat_issue(s):
        dstS_, mb_, sems_ = slots[s][3], slots[s][7], slots[s][11]
        pltpu.async_copy(mb_, aggsh.at[dstS_], sems_, add=True)

    def scat_wait(s):
        dstS_, mb_, sems_ = slots[s][3], slots[s][7], slots[s][11]
        pltpu.make_async_copy(mb_, aggsh.at[dstS_], sems_).wait()

    zero16 = jnp.zeros((16,), jnp.float32)

    def _zrow(r, _):
        for q in range(DH // 16):
            zb[r, pl.ds(q * 16, 16)] = zero16
        return 0
    lax.fori_loop(0, ZR, _zrow, 0)
    for t in range(RPT // ZR):
        r0 = sid * RPT + t * ZR
        pltpu.sync_copy(zb, aggsh.at[pl.ds(r0, ZR)])
    plsc.subcore_barrier()

    idx_issue(0, 0)
    idx_wait(0, 0)
    cidx(0)
    gather_issue(0)
    idx_issue(1, 1)

    def _pair(j, _):
        for s in (0, 1):
            k = 2 * j + s

            @pl.when(k + 1 < NCH)
            def _():
                idx_wait(k + 1, 1 - s)
                cidx(1 - s)
                gather_issue(1 - s)
            gather_wait(s)

            @pl.when(k >= 2)
            def _():
                scat_wait(s)
            dst_snap(s)

            @pl.when(k + 2 < NCH)
            def _():
                idx_issue(k + 2, s)
            compute(s)
            scat_issue(s)
        return 0

    lax.fori_loop(0, NCH // 2, _pair, 0)
    scat_wait(0)
    scat_wait(1)
    plsc.subcore_barrier()

    for t in range(RPT // ZR):
        r0 = sid * RPT + t * ZR
        pltpu.sync_copy(aggsh.at[pl.ds(r0, ZR)], out_hbm.at[cid, pl.ds(r0, ZR)])


_EDGE_CACHE = {}


def _build_edge_phase():
    return functools.partial(
        pl.kernel,
        out_type=jax.ShapeDtypeStruct((NC, NP, DH), jnp.float32),
        mesh=plsc.VectorSubcoreMesh(core_axis_name="c", subcore_axis_name="s",
                                    num_cores=NC, num_subcores=NS),
        compiler_params=pltpu.CompilerParams(use_tc_tiling_on_sc=False),
        scratch_types=(
            [pltpu.VMEM((C,), jnp.int32)] * 6
            + [pltpu.VMEM((C,), jnp.int32)] * 2
            + [pltpu.VMEM((C, DH), jnp.float32)] * 6
            + [pltpu.VMEM((ZR, DH), jnp.float32)]
            + [pltpu.VMEM_SHARED((NP, DH), jnp.float32)]
            + [pltpu.SemaphoreType.DMA] * 8
        ),
    )(_edge_body)


def _edge_phase(*args):
    if "f" not in _EDGE_CACHE:
        _EDGE_CACHE["f"] = _build_edge_phase()
    return _EDGE_CACHE["f"](*args)


def _mlp_body(h_ref, agg_ref, eps_ref, w1_ref, b1_ref, g1_ref, be1_ref,
              w2_ref, b2_ref, g_ref, be_ref, out_ref, *, last):
    hfull = jnp.concatenate([h_ref[0], h_ref[1]], axis=1)
    afull = jnp.concatenate([agg_ref[0, :N], agg_ref[1, :N]], axis=1)
    z = (1.0 + eps_ref[0]) * hfull + afull
    y = jnp.dot(z, w1_ref[...], preferred_element_type=jnp.float32)
    y = y + b1_ref[...]
    m = jnp.mean(y, axis=0, keepdims=True)
    v = jnp.mean((y - m) ** 2, axis=0, keepdims=True)
    y = (y - m) * lax.rsqrt(v + 1e-5) * g1_ref[...] + be1_ref[...]
    y = jnp.maximum(y, 0.0)
    o = jnp.dot(y, w2_ref[...], preferred_element_type=jnp.float32)
    o = o + b2_ref[...]
    m2 = jnp.mean(o, axis=0, keepdims=True)
    v2 = jnp.mean((o - m2) ** 2, axis=0, keepdims=True)
    o = (o - m2) * lax.rsqrt(v2 + 1e-5) * g_ref[...] + be_ref[...]
    if last:
        out_ref[...] = o
    else:
        o = jnp.maximum(o, 0.0)
        out_ref[0] = o[:, :DH]
        out_ref[1] = o[:, DH:]


def _mlp(last):
    return pl.pallas_call(
        functools.partial(_mlp_body, last=last),
        in_specs=[
            pl.BlockSpec(memory_space=pltpu.VMEM),
            pl.BlockSpec(memory_space=pltpu.VMEM),
            pl.BlockSpec(memory_space=pltpu.SMEM),
            pl.BlockSpec(memory_space=pltpu.VMEM),
            pl.BlockSpec(memory_space=pltpu.VMEM),
            pl.BlockSpec(memory_space=pltpu.VMEM),
            pl.BlockSpec(memory_space=pltpu.VMEM),
            pl.BlockSpec(memory_space=pltpu.VMEM),
            pl.BlockSpec(memory_space=pltpu.VMEM),
            pl.BlockSpec(memory_space=pltpu.VMEM),
            pl.BlockSpec(memory_space=pltpu.VMEM),
        ],
        out_shape=(jax.ShapeDtypeStruct((N, D), jnp.float32) if last
                   else jax.ShapeDtypeStruct((2, N, DH), jnp.float32)),
    )


def kernel(x, edge_index, edge_attr, batch, atom_emb, bond_emb, eps,
           W1, b1, g1, be1, W2, b2, g, be):
    h, T, cflat = _encode(x.astype(jnp.int32), atom_emb, bond_emb,
                          edge_attr.astype(jnp.int32).T)
    src = edge_index[0].astype(jnp.int32)
    dst = edge_index[1].astype(jnp.int32)
    for l in range(L):
        agg = _edge_phase(h.reshape(2 * N, DH), src, dst, cflat.reshape(E),
                          T[l].reshape(2 * 512, DH))
        h = _mlp(l == L - 1)(
            h, agg, eps[l].reshape(1),
            W1[l], b1[l].reshape(1, 2 * D), g1[l].reshape(1, 2 * D),
            be1[l].reshape(1, 2 * D),
            W2[l], b2[l].reshape(1, D), g[l].reshape(1, D),
            be[l].reshape(1, D))
    return (h, batch)
